# baseline (speedup 1.0000x reference)
import jax
import jax.numpy as jnp
from jax.experimental import pallas as pl
from jax.experimental.pallas import tpu as pltpu


def _dot(a, b):
    return jnp.dot(a, b, preferred_element_type=jnp.float32)


def _fused_kernel(x_ref, lu_ref, ld_ref, w1_ref, w23_ref, wfc_ref,
                  out_ref, y_ref, *, bm, it):
    g = pl.program_id(0)
    phase = g // it
    m = g % it
    rows = pl.ds(m * bm, bm)

    @pl.when(phase == 0)
    def _proj():
        x = x_ref[...]
        y_ref[0, 0, rows, :] = _dot(x, w1_ref[0])
        y_ref[0, 1, rows, :] = _dot(x, w1_ref[1])
        y_ref[0, 2, rows, :] = _dot(x, w1_ref[2])

    @pl.when((phase == 1) | (phase == 2))
    def _layer():
        src = (phase - 1) % 2
        dst = phase % 2
        z = jnp.tanh(_dot(lu_ref[...], y_ref[src, 0])
                     + _dot(ld_ref[...], y_ref[src, 1])
                     + y_ref[src, 2, rows, :])
        w = w23_ref[phase - 1]
        y_ref[dst, 0, rows, :] = _dot(z, w[0])
        y_ref[dst, 1, rows, :] = _dot(z, w[1])
        y_ref[dst, 2, rows, :] = _dot(z, w[2])

    @pl.when(phase == 3)
    def _final():
        z = jnp.tanh(_dot(lu_ref[...], y_ref[0, 0])
                     + _dot(ld_ref[...], y_ref[0, 1])
                     + y_ref[0, 2, rows, :])
        gv = _dot(z, wfc_ref[...])
        nrm = jnp.sqrt(jnp.sum(gv * gv, axis=1, keepdims=True))
        nrm = jnp.maximum(nrm, 1e-12)
        out_ref[...] = jnp.tanh(gv / nrm)


def kernel(X, L_u, L_d, W1u, W1d, W1i, W2u, W2d, W2i, W3u, W3d, W3i, Wfc):
    n, f = X.shape
    h = W1u.shape[1]
    o = Wfc.shape[1]
    bm = 80 if n % 80 == 0 else n
    it = n // bm

    W1 = jnp.stack([W1u, W1d, W1i])
    W23 = jnp.stack([jnp.stack([W2u, W2d, W2i]),
                     jnp.stack([W3u, W3d, W3i])])

    def l_idx(g):
        return (jnp.where(g < it, 0, g % it), 0)

    def x_idx(g):
        return (jnp.minimum(g, it - 1), 0)

    def out_idx(g):
        return (jnp.where(g < 3 * it, 0, g % it), 0)

    import functools
    fused = pl.pallas_call(
        functools.partial(_fused_kernel, bm=bm, it=it),
        grid=(4 * it,),
        in_specs=[
            pl.BlockSpec((bm, f), x_idx),
            pl.BlockSpec((bm, n), l_idx),
            pl.BlockSpec((bm, n), l_idx),
            pl.BlockSpec((3, f, h), lambda g: (0, 0, 0)),
            pl.BlockSpec((2, 3, h, h), lambda g: (0, 0, 0, 0)),
            pl.BlockSpec((h, o), lambda g: (0, 0)),
        ],
        out_specs=pl.BlockSpec((bm, o), out_idx),
        out_shape=jax.ShapeDtypeStruct((n, o), jnp.float32),
        scratch_shapes=[pltpu.VMEM((2, 3, n, h), jnp.float32)],
        compiler_params=pltpu.CompilerParams(
            dimension_semantics=("arbitrary",)),
    )
    return fused(X, L_u, L_d, W1, W23, Wfc)


# speedup vs baseline: 1.2042x; 1.2042x over previous
import jax
import jax.numpy as jnp
from jax.experimental import pallas as pl
from jax.experimental.pallas import tpu as pltpu


def _dot(a, b):
    return jnp.dot(a, b, preferred_element_type=jnp.float32)


def _fused_kernel(x_ref, lu_ref, ld_ref, w1_ref, w23_ref, wfc_ref,
                  out_ref, y_ref, *, bm, it):
    g = pl.program_id(0)
    phase = g // it
    m = g % it
    rows = pl.ds(m * bm, bm)

    @pl.when(phase == 0)
    def _proj():
        x = x_ref[...]
        y_ref[0, 0, rows, :] = _dot(x, w1_ref[0])
        y_ref[0, 1, rows, :] = _dot(x, w1_ref[1])
        y_ref[0, 2, rows, :] = _dot(x, w1_ref[2])

    @pl.when((phase == 1) | (phase == 2))
    def _layer():
        src = (phase - 1) % 2
        dst = phase % 2
        z = jnp.tanh(_dot(lu_ref[...], y_ref[src, 0])
                     + _dot(ld_ref[...], y_ref[src, 1])
                     + y_ref[src, 2, rows, :])
        w = w23_ref[phase - 1]
        y_ref[dst, 0, rows, :] = _dot(z, w[0])
        y_ref[dst, 1, rows, :] = _dot(z, w[1])
        y_ref[dst, 2, rows, :] = _dot(z, w[2])

    @pl.when(phase == 3)
    def _final():
        z = jnp.tanh(_dot(lu_ref[...], y_ref[0, 0])
                     + _dot(ld_ref[...], y_ref[0, 1])
                     + y_ref[0, 2, rows, :])
        gv = _dot(z, wfc_ref[...])
        nrm = jnp.sqrt(jnp.sum(gv * gv, axis=1, keepdims=True))
        nrm = jnp.maximum(nrm, 1e-12)
        out_ref[...] = jnp.tanh(gv / nrm)


def kernel(X, L_u, L_d, W1u, W1d, W1i, W2u, W2d, W2i, W3u, W3d, W3i, Wfc):
    n, f = X.shape
    h = W1u.shape[1]
    o = Wfc.shape[1]
    bm = 200 if n % 200 == 0 else n
    it = n // bm

    W1 = jnp.stack([W1u, W1d, W1i])
    W23 = jnp.stack([jnp.stack([W2u, W2d, W2i]),
                     jnp.stack([W3u, W3d, W3i])])

    def l_idx(g):
        return (jnp.where(g < it, 0, g % it), 0)

    def x_idx(g):
        return (jnp.minimum(g, it - 1), 0)

    def out_idx(g):
        return (jnp.where(g < 3 * it, 0, g % it), 0)

    import functools
    fused = pl.pallas_call(
        functools.partial(_fused_kernel, bm=bm, it=it),
        grid=(4 * it,),
        in_specs=[
            pl.BlockSpec((bm, f), x_idx),
            pl.BlockSpec((bm, n), l_idx),
            pl.BlockSpec((bm, n), l_idx),
            pl.BlockSpec((3, f, h), lambda g: (0, 0, 0)),
            pl.BlockSpec((2, 3, h, h), lambda g: (0, 0, 0, 0)),
            pl.BlockSpec((h, o), lambda g: (0, 0)),
        ],
        out_specs=pl.BlockSpec((bm, o), out_idx),
        out_shape=jax.ShapeDtypeStruct((n, o), jnp.float32),
        scratch_shapes=[pltpu.VMEM((2, 3, n, h), jnp.float32)],
        compiler_params=pltpu.CompilerParams(
            dimension_semantics=("arbitrary",),
            vmem_limit_bytes=64 * 1024 * 1024),
    )
    return fused(X, L_u, L_d, W1, W23, Wfc)


# speedup vs baseline: 1.2435x; 1.0327x over previous
import functools

import jax
import jax.numpy as jnp
from jax.experimental import pallas as pl
from jax.experimental.pallas import tpu as pltpu

_IT0 = 5


def _dot(a, b):
    return jnp.dot(a, b, preferred_element_type=jnp.float32)


def _fused_kernel(x_ref, lu_ref, ld_ref, w1_ref, w23_ref, wfc_ref,
                  out_ref, y_ref, *, bm, bm0, it):
    g = pl.program_id(0)
    q = g - _IT0
    phase = 1 + q // it
    m = q % it
    rows = pl.ds(m * bm, bm)

    @pl.when(g < _IT0)
    def _proj():
        rows0 = pl.ds(g * bm0, bm0)
        x = x_ref[...]
        y_ref[0, 0, rows0, :] = _dot(x, w1_ref[0])
        y_ref[0, 1, rows0, :] = _dot(x, w1_ref[1])
        y_ref[0, 2, rows0, :] = _dot(x, w1_ref[2])

    @pl.when((g >= _IT0) & (phase <= 2))
    def _layer():
        src = (phase - 1) % 2
        dst = phase % 2
        z = jnp.tanh(_dot(lu_ref[...], y_ref[src, 0])
                     + _dot(ld_ref[...], y_ref[src, 1])
                     + y_ref[src, 2, rows, :])
        w = w23_ref[phase - 1]
        y_ref[dst, 0, rows, :] = _dot(z, w[0])
        y_ref[dst, 1, rows, :] = _dot(z, w[1])
        y_ref[dst, 2, rows, :] = _dot(z, w[2])

    @pl.when(phase == 3)
    def _final():
        z = jnp.tanh(_dot(lu_ref[...], y_ref[0, 0])
                     + _dot(ld_ref[...], y_ref[0, 1])
                     + y_ref[0, 2, rows, :])
        gv = _dot(z, wfc_ref[...])
        nrm = jnp.sqrt(jnp.sum(gv * gv, axis=1, keepdims=True))
        nrm = jnp.maximum(nrm, 1e-12)
        out_ref[...] = jnp.tanh(gv / nrm)


def kernel(X, L_u, L_d, W1u, W1d, W1i, W2u, W2d, W2i, W3u, W3d, W3i, Wfc):
    n, f = X.shape
    h = W1u.shape[1]
    o = Wfc.shape[1]
    bm = 200 if n % 200 == 0 else n
    it = n // bm
    bm0 = n // _IT0

    W1 = jnp.stack([W1u, W1d, W1i])
    W23 = jnp.stack([jnp.stack([W2u, W2d, W2i]),
                     jnp.stack([W3u, W3d, W3i])])

    def l_idx(g):
        return (jnp.where(g < _IT0, 0, (g - _IT0) % it), 0)

    def x_idx(g):
        return (jnp.minimum(g, _IT0 - 1), 0)

    def out_idx(g):
        return (jnp.where(g < _IT0 + 2 * it, 0, (g - _IT0) % it), 0)

    fused = pl.pallas_call(
        functools.partial(_fused_kernel, bm=bm, bm0=bm0, it=it),
        grid=(_IT0 + 3 * it,),
        in_specs=[
            pl.BlockSpec((bm0, f), x_idx),
            pl.BlockSpec((bm, n), l_idx),
            pl.BlockSpec((bm, n), l_idx),
            pl.BlockSpec((3, f, h), lambda g: (0, 0, 0)),
            pl.BlockSpec((2, 3, h, h), lambda g: (0, 0, 0, 0)),
            pl.BlockSpec((h, o), lambda g: (0, 0)),
        ],
        out_specs=pl.BlockSpec((bm, o), out_idx),
        out_shape=jax.ShapeDtypeStruct((n, o), jnp.float32),
        scratch_shapes=[pltpu.VMEM((2, 3, n, h), jnp.float32)],
        compiler_params=pltpu.CompilerParams(
            dimension_semantics=("arbitrary",),
            vmem_limit_bytes=64 * 1024 * 1024),
    )
    return fused(X, L_u, L_d, W1, W23, Wfc)


# speedup vs baseline: 1.2526x; 1.0073x over previous
import functools

import jax
import jax.numpy as jnp
from jax.experimental import pallas as pl
from jax.experimental.pallas import tpu as pltpu

_IT0 = 5


def _dot(a, b):
    return jnp.dot(a, b, preferred_element_type=jnp.float32)


def _fused_kernel(x_ref, lu_ref, ld_ref, w1_ref, w23_ref, wfc_ref,
                  out_ref, y_ref, *, bm, bm0, it):
    g = pl.program_id(0)
    q = g - _IT0
    phase = 1 + q // it
    m = q % it
    rows = pl.ds(m * bm, bm)

    @pl.when(g < _IT0)
    def _proj():
        rows0 = pl.ds(g * bm0, bm0)
        x = x_ref[...]
        y_ref[0, 0, rows0, :] = _dot(x, w1_ref[0])
        y_ref[0, 1, rows0, :] = _dot(x, w1_ref[1])
        y_ref[0, 2, rows0, :] = _dot(x, w1_ref[2])

    def _layer_body(src, dst, wsel):
        z = jnp.tanh(_dot(lu_ref[...], y_ref[src, 0])
                     + _dot(ld_ref[...], y_ref[src, 1])
                     + y_ref[src, 2, rows, :])
        w = w23_ref[wsel]
        y_ref[dst, 0, rows, :] = _dot(z, w[0])
        y_ref[dst, 1, rows, :] = _dot(z, w[1])
        y_ref[dst, 2, rows, :] = _dot(z, w[2])

    @pl.when(phase == 1)
    def _layer1():
        _layer_body(0, 1, 0)

    @pl.when(phase == 2)
    def _layer2():
        _layer_body(1, 0, 1)

    @pl.when(phase == 3)
    def _final():
        z = jnp.tanh(_dot(lu_ref[...], y_ref[0, 0])
                     + _dot(ld_ref[...], y_ref[0, 1])
                     + y_ref[0, 2, rows, :])
        gv = _dot(z, wfc_ref[...])
        nrm = jnp.sqrt(jnp.sum(gv * gv, axis=1, keepdims=True))
        nrm = jnp.maximum(nrm, 1e-12)
        out_ref[...] = jnp.tanh(gv / nrm)


def kernel(X, L_u, L_d, W1u, W1d, W1i, W2u, W2d, W2i, W3u, W3d, W3i, Wfc):
    n, f = X.shape
    h = W1u.shape[1]
    o = Wfc.shape[1]
    bm = 200 if n % 200 == 0 else n
    it = n // bm
    bm0 = n // _IT0

    W1 = jnp.stack([W1u, W1d, W1i])
    W23 = jnp.stack([jnp.stack([W2u, W2d, W2i]),
                     jnp.stack([W3u, W3d, W3i])])

    def l_idx(g):
        return (jnp.where(g < _IT0, 0, (g - _IT0) % it), 0)

    def x_idx(g):
        return (jnp.minimum(g, _IT0 - 1), 0)

    def out_idx(g):
        return (jnp.where(g < _IT0 + 2 * it, 0, (g - _IT0) % it), 0)

    fused = pl.pallas_call(
        functools.partial(_fused_kernel, bm=bm, bm0=bm0, it=it),
        grid=(_IT0 + 3 * it,),
        in_specs=[
            pl.BlockSpec((bm0, f), x_idx),
            pl.BlockSpec((bm, n), l_idx),
            pl.BlockSpec((bm, n), l_idx),
            pl.BlockSpec((3, f, h), lambda g: (0, 0, 0)),
            pl.BlockSpec((2, 3, h, h), lambda g: (0, 0, 0, 0)),
            pl.BlockSpec((h, o), lambda g: (0, 0)),
        ],
        out_specs=pl.BlockSpec((bm, o), out_idx),
        out_shape=jax.ShapeDtypeStruct((n, o), jnp.float32),
        scratch_shapes=[pltpu.VMEM((2, 3, n, h), jnp.float32)],
        compiler_params=pltpu.CompilerParams(
            dimension_semantics=("arbitrary",),
            vmem_limit_bytes=64 * 1024 * 1024),
    )
    return fused(X, L_u, L_d, W1, W23, Wfc)
